# baseline (speedup 1.0000x reference)
import functools
import jax
import jax.numpy as jnp
from jax import lax
from jax.experimental import pallas as pl
from jax.experimental.pallas import tpu as pltpu
from jax.experimental.pallas import tpu_sc as plsc

_NUM_CB = 1024
_DIM = 32
_EPS = 1e-07
_N = 65536
_BLK = 2048
_GRID = _N // _BLK

_NC = 2
_NS = 16
_NW = _NC * _NS
_CHUNK = 128
_B_PER_W = _N // _NW
_NCHUNK = _B_PER_W // _CHUNK


def _vq_body(x_ref, cb_ref, oh_ref, idx_ref, perp_ref, counts_ref):
    i = pl.program_id(0)
    x = x_ref[...]
    cb = cb_ref[...]
    cbsq = jnp.sum(cb * cb, axis=-1, keepdims=True)
    c0 = jnp.bfloat16(cbsq).astype(jnp.float32)
    r1 = cbsq - c0
    c1 = jnp.bfloat16(r1).astype(jnp.float32)
    c2 = r1 - c1
    cb_aug = jnp.concatenate([cb * -2.0, c0, c1, c2], axis=1)
    ones_col = jnp.ones((_BLK, 1), jnp.float32)
    x_aug = jnp.concatenate([x, ones_col, ones_col, ones_col], axis=1)
    dist = jax.lax.dot_general(x_aug, cb_aug, (((1,), (1,)), ((), ())),
                               preferred_element_type=jnp.float32)
    minval = jnp.min(dist, axis=-1, keepdims=True)
    oh = jnp.where(dist == minval, 1.0, 0.0)
    oh_ref[...] = oh
    lane = jax.lax.broadcasted_iota(jnp.int32, (_NUM_CB, 1), 0).astype(jnp.float32)
    hi = jnp.bfloat16(lane).astype(jnp.float32)
    lo = lane - hi
    iota2 = jnp.concatenate([hi, lo], axis=1)
    idxf = jax.lax.dot_general(oh, iota2, (((1,), (0,)), ((), ())),
                               preferred_element_type=jnp.float32)
    idx_ref[...] = jnp.sum(idxf, axis=-1, keepdims=True).astype(jnp.int32)
    ones_row = jnp.ones((1, _BLK), dtype=jnp.float32)
    bc = jax.lax.dot_general(ones_row, oh, (((1,), (0,)), ((), ())),
                             preferred_element_type=jnp.float32)

    @pl.when(i == 0)
    def _():
        counts_ref[...] = bc

    @pl.when(i > 0)
    def _():
        counts_ref[...] = counts_ref[...] + bc

    @pl.when(i == _GRID - 1)
    def _():
        cnt = counts_ref[...]
        prob = cnt / jnp.sum(cnt)
        perp = jnp.exp(-jnp.sum(prob * jnp.log(prob + _EPS)))
        perp_ref[...] = jnp.full((1, 1), perp, dtype=jnp.float32)


_sc_mesh = plsc.VectorSubcoreMesh(core_axis_name="c", subcore_axis_name="s")


@functools.partial(
    pl.kernel,
    mesh=_sc_mesh,
    out_type=jax.ShapeDtypeStruct((_N, 128), jnp.float32),
    scratch_types=[
        pltpu.VMEM((_NCHUNK, _CHUNK), jnp.int32),
        pltpu.VMEM((512, 128), jnp.float32),
        pltpu.SemaphoreType.DMA,
    ],
)
def _sc_gather(idx_hbm, table_hbm, out_hbm, idx_v, rows_v, sem):
    wid = lax.axis_index("s") * _NC + lax.axis_index("c")
    pltpu.sync_copy(idx_hbm.at[wid], idx_v)
    for g in range(4):
        copies = []
        for j in range(4):
            copies.append(pltpu.async_copy(
                table_hbm.at[idx_v.at[g * 4 + j]],
                rows_v.at[pl.ds(j * _CHUNK, _CHUNK)], sem))
        for c in copies:
            c.wait()
        pltpu.sync_copy(
            rows_v, out_hbm.at[pl.ds(wid * _B_PER_W + g * 512, 512)])


def kernel(flat_x, codebook):
    oh, idx, perp = pl.pallas_call(
        _vq_body,
        grid=(_GRID,),
        in_specs=[
            pl.BlockSpec((_BLK, _DIM), lambda i: (i, 0)),
            pl.BlockSpec((_NUM_CB, _DIM), lambda i: (0, 0)),
        ],
        out_specs=[
            pl.BlockSpec((_BLK, _NUM_CB), lambda i: (i, 0)),
            pl.BlockSpec((_BLK, 1), lambda i: (i, 0)),
            pl.BlockSpec((1, 1), lambda i: (0, 0)),
        ],
        out_shape=[
            jax.ShapeDtypeStruct((_N, _NUM_CB), jnp.float32),
            jax.ShapeDtypeStruct((_N, 1), jnp.int32),
            jax.ShapeDtypeStruct((1, 1), jnp.float32),
        ],
        scratch_shapes=[pltpu.VMEM((1, _NUM_CB), jnp.float32)],
    )(flat_x, codebook)
    idx3 = idx.reshape(_NW, _NCHUNK, _CHUNK)
    cb_pad = jnp.pad(codebook, ((0, 0), (0, 128 - _DIM)))
    xq = _sc_gather(idx3, cb_pad)[:, :_DIM]
    return (xq, oh, perp[0, 0])


# speedup vs baseline: 1.6968x; 1.6968x over previous
import jax
import jax.numpy as jnp
from jax.experimental import pallas as pl
from jax.experimental.pallas import tpu as pltpu

_NUM_CB = 1024
_DIM = 32
_EPS = 1e-07
_N = 65536
_BLK = 2048
_GRID = _N // _BLK


def _vq_body(x_ref, cb_ref, xq_ref, oh_ref, perp_ref, counts_ref):
    i = pl.program_id(0)
    x = x_ref[...]
    cb = cb_ref[...]
    cbsq = jnp.sum(cb * cb, axis=-1, keepdims=True)
    c0 = jnp.bfloat16(cbsq).astype(jnp.float32)
    r1 = cbsq - c0
    c1 = jnp.bfloat16(r1).astype(jnp.float32)
    c2 = r1 - c1
    cb_aug = jnp.concatenate([cb * -2.0, c0, c1, c2], axis=1)
    ones_col = jnp.ones((_BLK, 1), jnp.float32)
    x_aug = jnp.concatenate([x, ones_col, ones_col, ones_col], axis=1)
    dist = jax.lax.dot_general(x_aug, cb_aug, (((1,), (1,)), ((), ())),
                               preferred_element_type=jnp.float32)
    minval = jnp.min(dist, axis=-1, keepdims=True)
    oh = jnp.where(dist == minval, 1.0, 0.0)
    oh_ref[...] = oh
    xq_ref[...] = jax.lax.dot_general(oh, cb, (((1,), (0,)), ((), ())),
                                      preferred_element_type=jnp.float32)
    ones_row = jnp.ones((1, _BLK), dtype=jnp.float32)
    bc = jax.lax.dot_general(ones_row, oh, (((1,), (0,)), ((), ())),
                             preferred_element_type=jnp.float32)

    @pl.when(i == 0)
    def _():
        counts_ref[...] = bc

    @pl.when(i > 0)
    def _():
        counts_ref[...] = counts_ref[...] + bc

    @pl.when(i == _GRID - 1)
    def _():
        cnt = counts_ref[...]
        prob = cnt / jnp.sum(cnt)
        perp = jnp.exp(-jnp.sum(prob * jnp.log(prob + _EPS)))
        perp_ref[...] = jnp.full((1, 1), perp, dtype=jnp.float32)


def kernel(flat_x, codebook):
    xq, oh, perp = pl.pallas_call(
        _vq_body,
        grid=(_GRID,),
        in_specs=[
            pl.BlockSpec((_BLK, _DIM), lambda i: (i, 0)),
            pl.BlockSpec((_NUM_CB, _DIM), lambda i: (0, 0)),
        ],
        out_specs=[
            pl.BlockSpec((_BLK, _DIM), lambda i: (i, 0)),
            pl.BlockSpec((_BLK, _NUM_CB), lambda i: (i, 0)),
            pl.BlockSpec((1, 1), lambda i: (0, 0)),
        ],
        out_shape=[
            jax.ShapeDtypeStruct((_N, _DIM), jnp.float32),
            jax.ShapeDtypeStruct((_N, _NUM_CB), jnp.float32),
            jax.ShapeDtypeStruct((1, 1), jnp.float32),
        ],
        scratch_shapes=[pltpu.VMEM((1, _NUM_CB), jnp.float32)],
    )(flat_x, codebook)
    return (xq, oh, perp[0, 0])
